# initial kernel scaffold (unmeasured)
import jax
import jax.numpy as jnp
from jax import lax
from jax.experimental import pallas as pl
from jax.experimental.pallas import tpu as pltpu

T = 2048
D = 4096
V = 16384
V_HALF = V // 2


def _exchange(logits):

    def body(logits_ref, out_ref, local_sem, send_sem, recv_sem):
        my_x = lax.axis_index("x")
        my_y = lax.axis_index("y")
        partner = (1 - my_x, my_y)
        my_col = my_x * V_HALF

        lcopy = pltpu.make_async_copy(
            logits_ref, out_ref.at[:, pl.ds(my_col, V_HALF)], local_sem
        )
        lcopy.start()

        barrier = pltpu.get_barrier_semaphore()
        pl.semaphore_signal(
            barrier, inc=1, device_id=partner,
            device_id_type=pl.DeviceIdType.MESH,
        )
        pl.semaphore_wait(barrier, 1)

        rdma = pltpu.make_async_remote_copy(
            src_ref=logits_ref,
            dst_ref=out_ref.at[:, pl.ds(my_col, V_HALF)],
            send_sem=send_sem,
            recv_sem=recv_sem,
            device_id=partner,
            device_id_type=pl.DeviceIdType.MESH,
        )
        rdma.start()
        rdma.wait()
        lcopy.wait()

    return pl.pallas_call(
        body,
        out_shape=jax.ShapeDtypeStruct((T, V), jnp.float32),
        in_specs=[pl.BlockSpec(memory_space=pltpu.ANY)],
        out_specs=pl.BlockSpec(memory_space=pltpu.ANY),
        scratch_shapes=[
            pltpu.SemaphoreType.DMA,
            pltpu.SemaphoreType.DMA,
            pltpu.SemaphoreType.DMA,
        ],
        compiler_params=pltpu.CompilerParams(collective_id=0),
    )(logits)


def _softmax(full):
    BT = 128

    def body(in_ref, out_ref):
        z = in_ref[...]
        m = jnp.max(z, axis=-1, keepdims=True)
        e = jnp.exp(z - m)
        out_ref[...] = e / jnp.sum(e, axis=-1, keepdims=True)

    return pl.pallas_call(
        body,
        out_shape=jax.ShapeDtypeStruct((T, V), jnp.float32),
        grid=(T // BT,),
        in_specs=[pl.BlockSpec((BT, V), lambda i: (i, 0))],
        out_specs=pl.BlockSpec((BT, V), lambda i: (i, 0)),
    )(full)


def kernel(x, W):
    logits = jnp.dot(x, W, preferred_element_type=jnp.float32)
    full = _exchange(logits)
    return _softmax(full)


# baseline (device time: 2324478 ns/iter reference)
import jax
import jax.numpy as jnp
from jax import lax
from jax.experimental import pallas as pl
from jax.experimental.pallas import tpu as pltpu

T = 2048
D = 4096
V = 16384
V_HALF = V // 2


def _exchange(logits):

    def body(logits_ref, out_ref, local_sem, send_sem, recv_sem):
        my_x = lax.axis_index("x")
        my_y = lax.axis_index("y")
        partner = (1 - my_x, my_y)
        my_col = my_x * V_HALF

        lcopy = pltpu.make_async_copy(
            logits_ref, out_ref.at[:, pl.ds(my_col, V_HALF)], local_sem
        )
        lcopy.start()

        barrier = pltpu.get_barrier_semaphore()
        pl.semaphore_signal(
            barrier, inc=1, device_id=partner,
            device_id_type=pl.DeviceIdType.MESH,
        )
        pl.semaphore_wait(barrier, 1)

        rdma = pltpu.make_async_remote_copy(
            src_ref=logits_ref,
            dst_ref=out_ref.at[:, pl.ds(my_col, V_HALF)],
            send_sem=send_sem,
            recv_sem=recv_sem,
            device_id=partner,
            device_id_type=pl.DeviceIdType.MESH,
        )
        rdma.start()
        rdma.wait()
        lcopy.wait()

    return pl.pallas_call(
        body,
        out_shape=jax.ShapeDtypeStruct((T, V), jnp.float32),
        in_specs=[pl.BlockSpec(memory_space=pl.ANY)],
        out_specs=pl.BlockSpec(memory_space=pl.ANY),
        scratch_shapes=[
            pltpu.SemaphoreType.DMA,
            pltpu.SemaphoreType.DMA,
            pltpu.SemaphoreType.DMA,
        ],
        compiler_params=pltpu.CompilerParams(collective_id=0),
    )(logits)


def _softmax(full):
    BT = 64

    def body(in_ref, out_ref):
        z = in_ref[...]
        m = jnp.max(z, axis=-1, keepdims=True)
        e = jnp.exp(z - m)
        out_ref[...] = e / jnp.sum(e, axis=-1, keepdims=True)

    return pl.pallas_call(
        body,
        out_shape=jax.ShapeDtypeStruct((T, V), jnp.float32),
        grid=(T // BT,),
        in_specs=[pl.BlockSpec((BT, V), lambda i: (i, 0))],
        out_specs=pl.BlockSpec((BT, V), lambda i: (i, 0)),
        compiler_params=pltpu.CompilerParams(
            vmem_limit_bytes=100 * 1024 * 1024
        ),
    )(full)


def kernel(x, W):
    logits = jnp.dot(x, W, preferred_element_type=jnp.float32)
    full = _exchange(logits)
    return _softmax(full)


# device time: 517016 ns/iter; 4.4959x vs baseline; 4.4959x over previous
import jax
import jax.numpy as jnp
from jax import lax
from jax.experimental import pallas as pl
from jax.experimental.pallas import tpu as pltpu

T = 2048
D = 4096
V = 16384
V_HALF = V // 2


ROWS_HALF = T // 2
NCF = 16
RC = ROWS_HALF // NCF


def _exchange(logits):

    comm_dtype = logits.dtype

    def body(logits_ref, other_ref, sx, rx, sy, ry):
        my_x = lax.axis_index("x")
        my_y = lax.axis_index("y")
        xp = (1 - my_x, my_y)
        yp = (my_x, 1 - my_y)
        base = my_y * ROWS_HALF
        obase = (1 - my_y) * ROWS_HALF

        barrier = pltpu.get_barrier_semaphore()
        for nbr in (xp, yp):
            pl.semaphore_signal(
                barrier, inc=1, device_id=nbr,
                device_id_type=pl.DeviceIdType.MESH,
            )
        pl.semaphore_wait(barrier, 2)

        x_rdmas = []
        for k in range(NCF):
            r0 = base + k * RC
            rd = pltpu.make_async_remote_copy(
                src_ref=logits_ref.at[pl.ds(r0, RC), :],
                dst_ref=other_ref.at[pl.ds(r0, RC), :],
                send_sem=sx.at[k], recv_sem=rx.at[k],
                device_id=xp, device_id_type=pl.DeviceIdType.MESH,
            )
            rd.start()
            x_rdmas.append(rd)

        y_rdmas = []
        for k in range(NCF):
            x_rdmas[k].wait_recv()
            r0 = base + k * RC
            fw = pltpu.make_async_remote_copy(
                src_ref=other_ref.at[pl.ds(r0, RC), :],
                dst_ref=other_ref.at[pl.ds(r0, RC), :],
                send_sem=sy.at[k], recv_sem=ry.at[k],
                device_id=yp, device_id_type=pl.DeviceIdType.MESH,
            )
            fw.start()
            y_rdmas.append(fw)

        for k in range(NCF):
            r0 = obase + k * RC
            pltpu.make_async_remote_copy(
                src_ref=other_ref.at[pl.ds(r0, RC), :],
                dst_ref=other_ref.at[pl.ds(r0, RC), :],
                send_sem=sy.at[k], recv_sem=ry.at[k],
                device_id=yp, device_id_type=pl.DeviceIdType.MESH,
            ).wait_recv()

        for k in range(NCF):
            x_rdmas[k].wait_send()
            y_rdmas[k].wait_send()

    return pl.pallas_call(
        body,
        out_shape=jax.ShapeDtypeStruct((T, V_HALF), comm_dtype),
        in_specs=[pl.BlockSpec(memory_space=pl.ANY)],
        out_specs=pl.BlockSpec(memory_space=pl.ANY),
        scratch_shapes=[
            pltpu.SemaphoreType.DMA((NCF,)),
            pltpu.SemaphoreType.DMA((NCF,)),
            pltpu.SemaphoreType.DMA((NCF,)),
            pltpu.SemaphoreType.DMA((NCF,)),
        ],
        compiler_params=pltpu.CompilerParams(collective_id=0),
    )(logits)


def _softmax(local, other):
    BT = 64

    def body(l_ref, o_ref, out_ref):
        my_x = lax.axis_index("x")
        zl = l_ref[...]
        zo = o_ref[...].astype(jnp.float32)
        m = jnp.maximum(
            jnp.max(zl, axis=-1, keepdims=True),
            jnp.max(zo, axis=-1, keepdims=True),
        )
        el = jnp.exp(zl - m)
        eo = jnp.exp(zo - m)
        s = (
            jnp.sum(el, axis=-1, keepdims=True)
            + jnp.sum(eo, axis=-1, keepdims=True)
        )
        pl_ = el / s
        po = eo / s

        @pl.when(my_x == 0)
        def _():
            out_ref[:, :V_HALF] = pl_
            out_ref[:, V_HALF:] = po

        @pl.when(my_x == 1)
        def _():
            out_ref[:, :V_HALF] = po
            out_ref[:, V_HALF:] = pl_

    return pl.pallas_call(
        body,
        out_shape=jax.ShapeDtypeStruct((T, V), jnp.float32),
        grid=(T // BT,),
        in_specs=[
            pl.BlockSpec((BT, V_HALF), lambda i: (i, 0)),
            pl.BlockSpec((BT, V_HALF), lambda i: (i, 0)),
        ],
        out_specs=pl.BlockSpec((BT, V), lambda i: (i, 0)),
        compiler_params=pltpu.CompilerParams(
            vmem_limit_bytes=100 * 1024 * 1024
        ),
    )(local, other)


def kernel(x, W):
    logits = jnp.dot(x, W, preferred_element_type=jnp.float32)
    other = _exchange(logits.astype(jnp.bfloat16))
    return _softmax(logits, other)


# device time: 294916 ns/iter; 7.8818x vs baseline; 1.7531x over previous
import jax
import jax.numpy as jnp
from jax import lax
from jax.experimental import pallas as pl
from jax.experimental.pallas import tpu as pltpu

T = 2048
D = 4096
V = 16384
V_HALF = V // 2

ROWS_HALF = T // 2
BN = 256
NCH = V_HALF // BN
NSLOT = 4
LAG = 2


def _fused_gemm_exchange(x, W):

    def body(x_ref, w_ref, out_ref, other16_ref, staging, landing,
             sx, rx, sy, ry, dsem, credit):
        c = pl.program_id(0)
        my_x = lax.axis_index("x")
        my_y = lax.axis_index("y")
        xp = (1 - my_x, my_y)
        yp = (my_x, 1 - my_y)
        base = my_y * ROWS_HALF
        obase = (1 - my_y) * ROWS_HALF

        def xsend_desc(k):
            s = k % NSLOT
            return pltpu.make_async_remote_copy(
                src_ref=staging.at[s],
                dst_ref=landing.at[s],
                send_sem=sx.at[k], recv_sem=rx.at[k],
                device_id=xp, device_id_type=pl.DeviceIdType.MESH,
            )

        def fwd_desc(k):
            return pltpu.make_async_remote_copy(
                src_ref=landing.at[k % NSLOT],
                dst_ref=other16_ref.at[pl.ds(base, ROWS_HALF),
                                       pl.ds(k * BN, BN)],
                send_sem=sy.at[k], recv_sem=ry.at[k],
                device_id=yp, device_id_type=pl.DeviceIdType.MESH,
            )

        def ywait_desc(k):
            return pltpu.make_async_remote_copy(
                src_ref=landing.at[k % NSLOT],
                dst_ref=other16_ref.at[pl.ds(obase, ROWS_HALF),
                                       pl.ds(k * BN, BN)],
                send_sem=sy.at[k], recv_sem=ry.at[k],
                device_id=yp, device_id_type=pl.DeviceIdType.MESH,
            )

        def drain_desc(k):
            return pltpu.make_async_copy(
                landing.at[k % NSLOT],
                other16_ref.at[pl.ds(base, ROWS_HALF), pl.ds(k * BN, BN)],
                dsem.at[k],
            )

        @pl.when(c == 0)
        def _():
            barrier = pltpu.get_barrier_semaphore()
            for nbr in (xp, yp):
                pl.semaphore_signal(
                    barrier, inc=1, device_id=nbr,
                    device_id_type=pl.DeviceIdType.MESH,
                )
            pl.semaphore_wait(barrier, 2)

        acc = jnp.dot(
            x_ref[...], w_ref[...], preferred_element_type=jnp.float32
        )
        acc16 = acc.astype(jnp.bfloat16)
        out_ref[...] = acc16

        @pl.when(c >= NSLOT)
        def _():
            xsend_desc(c - NSLOT).wait_send()

        s = c % NSLOT

        @pl.when(my_y == 0)
        def _():
            staging[s] = acc16[:ROWS_HALF]

        @pl.when(my_y == 1)
        def _():
            staging[s] = acc16[ROWS_HALF:]

        @pl.when(c >= LAG)
        def _():
            j = c - LAG
            xsend_desc(j).wait_recv()
            fwd_desc(j).start()
            drain_desc(j).start()

        @pl.when(c >= LAG + 2)
        def _():
            jj = c - LAG - 2
            fwd_desc(jj).wait_send()
            drain_desc(jj).wait()
            pl.semaphore_signal(
                credit, inc=1, device_id=xp,
                device_id_type=pl.DeviceIdType.MESH,
            )

        @pl.when(c >= NSLOT)
        def _():
            pl.semaphore_wait(credit, 1)
        xsend_desc(c).start()

        @pl.when(c == NCH - 1)
        def _():
            for j in range(NCH - LAG, NCH):
                xsend_desc(j).wait_recv()
                fwd_desc(j).start()
                drain_desc(j).start()
            for jj in range(NCH - LAG - 2, NCH):
                fwd_desc(jj).wait_send()
                drain_desc(jj).wait()
            for k in range(NCH - NSLOT, NCH):
                xsend_desc(k).wait_send()
            for k in range(NCH):
                ywait_desc(k).wait_recv()

    return pl.pallas_call(
        body,
        out_shape=(
            jax.ShapeDtypeStruct((T, V_HALF), jnp.bfloat16),
            jax.ShapeDtypeStruct((T, V_HALF), jnp.bfloat16),
        ),
        grid=(NCH,),
        in_specs=[
            pl.BlockSpec((T, D), lambda c: (0, 0)),
            pl.BlockSpec((D, BN), lambda c: (0, c)),
        ],
        out_specs=(
            pl.BlockSpec((T, BN), lambda c: (0, c)),
            pl.BlockSpec(memory_space=pl.ANY),
        ),
        scratch_shapes=[
            pltpu.VMEM((NSLOT, ROWS_HALF, BN), jnp.bfloat16),
            pltpu.VMEM((NSLOT, ROWS_HALF, BN), jnp.bfloat16),
            pltpu.SemaphoreType.DMA((NCH,)),
            pltpu.SemaphoreType.DMA((NCH,)),
            pltpu.SemaphoreType.DMA((NCH,)),
            pltpu.SemaphoreType.DMA((NCH,)),
            pltpu.SemaphoreType.DMA((NCH,)),
            pltpu.SemaphoreType.REGULAR,
        ],
        compiler_params=pltpu.CompilerParams(
            collective_id=0,
            vmem_limit_bytes=100 * 1024 * 1024,
        ),
    )(x, W)


def _softmax(local, other):
    BT = 64

    def body(l_ref, o_ref, out_ref):
        my_x = lax.axis_index("x")
        zl = l_ref[...].astype(jnp.float32)
        zo = o_ref[...].astype(jnp.float32)
        m = jnp.maximum(
            jnp.max(zl, axis=-1, keepdims=True),
            jnp.max(zo, axis=-1, keepdims=True),
        )
        el = jnp.exp(zl - m)
        eo = jnp.exp(zo - m)
        s = (
            jnp.sum(el, axis=-1, keepdims=True)
            + jnp.sum(eo, axis=-1, keepdims=True)
        )
        pl_ = el / s
        po = eo / s

        @pl.when(my_x == 0)
        def _():
            out_ref[:, :V_HALF] = pl_
            out_ref[:, V_HALF:] = po

        @pl.when(my_x == 1)
        def _():
            out_ref[:, :V_HALF] = po
            out_ref[:, V_HALF:] = pl_

    return pl.pallas_call(
        body,
        out_shape=jax.ShapeDtypeStruct((T, V), jnp.float32),
        grid=(T // BT,),
        in_specs=[
            pl.BlockSpec((BT, V_HALF), lambda i: (i, 0)),
            pl.BlockSpec((BT, V_HALF), lambda i: (i, 0)),
        ],
        out_specs=pl.BlockSpec((BT, V), lambda i: (i, 0)),
        compiler_params=pltpu.CompilerParams(
            vmem_limit_bytes=100 * 1024 * 1024
        ),
    )(local, other)


def kernel(x, W):
    logits, other16 = _fused_gemm_exchange(x, W)
    return _softmax(logits, other16)
